# Initial kernel scaffold; baseline (speedup 1.0000x reference)
import jax
import jax.numpy as jnp
from jax.experimental import pallas as pl


def kernel(q, k):
    raise NotImplementedError("write your pallas kernel here")


# speedup vs baseline: 1.6203x; 1.6203x over previous
import math

import jax
import jax.numpy as jnp
from jax.experimental import pallas as pl

B, L, S, H, D = 2, 8192, 128, 16, 64
TOPK = 16
BLOCK_SIZE = 64
WINDOW = 64
SM_SCALE = 1.0 / math.sqrt(D)

LT = 512


def _topk_kernel(q_ref, k_ref, out_s_ref, out_i_ref):
    l_idx = pl.program_id(1)
    tq = l_idx * LT + jax.lax.broadcasted_iota(jnp.int32, (LT, S), 0)
    ts = jax.lax.broadcasted_iota(jnp.int32, (LT, S), 1)
    allowed = (ts * BLOCK_SIZE + (BLOCK_SIZE + WINDOW - 1)) <= tq
    neg_inf = jnp.float32(-jnp.inf)

    for h in range(H):
        q_tile = q_ref[0, :, h, :]
        k_tile = k_ref[0, :, h, :]
        scores = jax.lax.dot_general(
            q_tile, k_tile,
            dimension_numbers=(((1,), (1,)), ((), ())),
            preferred_element_type=jnp.float32,
        ) * SM_SCALE
        s = jnp.where(allowed, scores, neg_inf)

        vals = []
        idxs = []
        for _ in range(TOPK):
            m = jnp.max(s, axis=1, keepdims=True)
            a = jnp.argmax(s, axis=1).astype(jnp.int32)
            vals.append(m)
            idxs.append(a[:, None])
            s = jnp.where(ts == a[:, None], neg_inf, s)

        top_s = jnp.concatenate(vals, axis=1)
        top_i = jnp.concatenate(idxs, axis=1)
        top_i = jnp.where(top_s == neg_inf, jnp.int32(-1), top_i)

        out_s_ref[0, :, h * TOPK:(h + 1) * TOPK] = top_s
        out_i_ref[0, :, h * TOPK:(h + 1) * TOPK] = top_i


@jax.jit
def kernel(q, k):
    grid = (B, L // LT)
    out_shape = (
        jax.ShapeDtypeStruct((B, L, H * TOPK), jnp.float32),
        jax.ShapeDtypeStruct((B, L, H * TOPK), jnp.int32),
    )
    q_spec = pl.BlockSpec((1, LT, H, D), lambda b, l: (b, l, 0, 0))
    k_spec = pl.BlockSpec((1, S, H, D), lambda b, l: (b, 0, 0, 0))
    o_spec = pl.BlockSpec((1, LT, H * TOPK), lambda b, l: (b, l, 0))
    flat_s, flat_i = pl.pallas_call(
        _topk_kernel,
        grid=grid,
        in_specs=[q_spec, k_spec],
        out_specs=(o_spec, o_spec),
        out_shape=out_shape,
    )(q, k)
    return (flat_s.reshape(B, L, H, TOPK), flat_i.reshape(B, L, H, TOPK))


# speedup vs baseline: 5.6919x; 3.5129x over previous
import math

import jax
import jax.numpy as jnp
from jax.experimental import pallas as pl
from jax.experimental.pallas import tpu as pltpu

B, L, S, H, D = 2, 8192, 128, 16, 64
TOPK = 16
BLOCK_SIZE = 64
WINDOW = 64
SM_SCALE = 1.0 / math.sqrt(D)

LT = 1024
NC = LT // 128
NG = S // TOPK


def _batcher_pairs(n):
    pairs = []
    p = 1
    while p < n:
        k = p
        while k >= 1:
            for j in range(k % p, n - k, 2 * k):
                for i in range(0, min(k, n - j - k)):
                    if (i + j) // (2 * p) == (i + j + k) // (2 * p):
                        pairs.append((i + j, i + j + k))
            k //= 2
        p *= 2
    return pairs


_PAIRS16 = _batcher_pairs(TOPK)


def _ce(v, i, a, b):
    m = v[a] >= v[b]
    va = jnp.where(m, v[a], v[b])
    ia = jnp.where(m, i[a], i[b])
    vb = jnp.where(m, v[b], v[a])
    ib = jnp.where(m, i[b], i[a])
    v[a], i[a], v[b], i[b] = va, ia, vb, ib


def _merge_top16(av, ai, bv, bi):
    cv, ci = [], []
    for t in range(TOPK):
        m = av[t] >= bv[TOPK - 1 - t]
        cv.append(jnp.where(m, av[t], bv[TOPK - 1 - t]))
        ci.append(jnp.where(m, ai[t], bi[TOPK - 1 - t]))
    for d in (8, 4, 2, 1):
        for t in range(TOPK):
            if t & d == 0:
                _ce(cv, ci, t, t + d)
    return cv, ci


def _topk_kernel(q_ref, k_ref, ov_ref, oi_ref, sc_ref):
    l_idx = pl.program_id(1)
    h = pl.program_id(2)
    tq = (l_idx * LT
          + 128 * jax.lax.broadcasted_iota(jnp.int32, (NC, 128), 0)
          + jax.lax.broadcasted_iota(jnp.int32, (NC, 128), 1))
    neg_inf = jnp.float32(-jnp.inf)

    q_h = q_ref[0, :, h, :]
    k_h = k_ref[0, :, h, :]
    res = jax.lax.dot_general(
        k_h, q_h,
        dimension_numbers=(((1,), (1,)), ((), ())),
        preferred_element_type=jnp.float32,
    )
    for j in range(NC):
        sc_ref[:, j, :] = res[:, j * 128:(j + 1) * 128]

    runs = []
    for g in range(NG):
        v, i = [], []
        for t in range(TOPK):
            s = TOPK * g + t
            pv = sc_ref[s]
            pv = jnp.where(tq >= BLOCK_SIZE * s + (BLOCK_SIZE + WINDOW - 1),
                           pv, neg_inf)
            v.append(pv)
            i.append(jnp.full((NC, 128), s, jnp.int32))
        for a, b in _PAIRS16:
            _ce(v, i, a, b)
        runs.append((v, i))
    while len(runs) > 1:
        runs = [_merge_top16(runs[a][0], runs[a][1], runs[a + 1][0], runs[a + 1][1])
                for a in range(0, len(runs), 2)]

    fv, fi = runs[0]
    for t in range(TOPK):
        vt = fv[t]
        it = jnp.where(vt == neg_inf, jnp.int32(-1), fi[t])
        ov_ref[0, 0, h * TOPK + t] = vt
        oi_ref[0, 0, h * TOPK + t] = it


@jax.jit
def kernel(q, k):
    k_scaled = k * jnp.float32(SM_SCALE)
    nl = L // LT
    grid = (B, nl, H)
    out_shape = (
        jax.ShapeDtypeStruct((B, nl, H * TOPK, NC, 128), jnp.float32),
        jax.ShapeDtypeStruct((B, nl, H * TOPK, NC, 128), jnp.int32),
    )
    q_spec = pl.BlockSpec((1, LT, H, D), lambda b, l, h: (b, l, 0, 0))
    k_spec = pl.BlockSpec((1, S, H, D), lambda b, l, h: (b, 0, 0, 0))
    o_spec = pl.BlockSpec((1, 1, H * TOPK, NC, 128), lambda b, l, h: (b, l, 0, 0, 0))
    ov, oi = pl.pallas_call(
        _topk_kernel,
        grid=grid,
        in_specs=[q_spec, k_spec],
        out_specs=(o_spec, o_spec),
        out_shape=out_shape,
        scratch_shapes=[pltpu.VMEM((S, NC, 128), jnp.float32)],
    )(q, k_scaled)
    ov = ov.transpose(0, 1, 3, 4, 2).reshape(B, L, H, TOPK)
    oi = oi.transpose(0, 1, 3, 4, 2).reshape(B, L, H, TOPK)
    return ov, oi


# speedup vs baseline: 6.3597x; 1.1173x over previous
import math

import jax
import jax.numpy as jnp
from jax.experimental import pallas as pl
from jax.experimental.pallas import tpu as pltpu

B, L, S, H, D = 2, 8192, 128, 16, 64
TOPK = 16
BLOCK_SIZE = 64
WINDOW = 64
SM_SCALE = 1.0 / math.sqrt(D)

LT = 1024
NC = LT // 128
NG = S // TOPK


def _batcher_pairs(n):
    pairs = []
    p = 1
    while p < n:
        k = p
        while k >= 1:
            for j in range(k % p, n - k, 2 * k):
                for i in range(0, min(k, n - j - k)):
                    if (i + j) // (2 * p) == (i + j + k) // (2 * p):
                        pairs.append((i + j, i + j + k))
            k //= 2
        p *= 2
    return pairs


_PAIRS16 = _batcher_pairs(TOPK)


def _ce(v, i, a, b):
    m = v[a] >= v[b]
    va = jnp.where(m, v[a], v[b])
    ia = jnp.where(m, i[a], i[b])
    vb = jnp.where(m, v[b], v[a])
    ib = jnp.where(m, i[b], i[a])
    v[a], i[a], v[b], i[b] = va, ia, vb, ib


def _merge_top16(av, ai, bv, bi):
    cv, ci = [], []
    for t in range(TOPK):
        m = av[t] >= bv[TOPK - 1 - t]
        cv.append(jnp.where(m, av[t], bv[TOPK - 1 - t]))
        ci.append(jnp.where(m, ai[t], bi[TOPK - 1 - t]))
    for d in (8, 4, 2, 1):
        for t in range(TOPK):
            if t & d == 0:
                _ce(cv, ci, t, t + d)
    return cv, ci


def _topk_kernel(q_ref, k_ref, ov_ref, oi_ref, sc_ref):
    l_idx = pl.program_id(1)
    h = pl.program_id(2)
    tq = (l_idx * LT
          + 128 * jax.lax.broadcasted_iota(jnp.int32, (NC, 128), 0)
          + jax.lax.broadcasted_iota(jnp.int32, (NC, 128), 1))
    neg_inf = jnp.float32(-jnp.inf)

    q_h = q_ref[0, :, h, :]
    k_h = k_ref[0, :, h, :]
    res = jax.lax.dot_general(
        k_h, q_h,
        dimension_numbers=(((1,), (1,)), ((), ())),
        preferred_element_type=jnp.float32,
    )
    for j in range(NC):
        sc_ref[:, j, :] = res[:, j * 128:(j + 1) * 128]

    def sorted_group(g):
        v, i = [], []
        for t in range(TOPK):
            s = g * TOPK + t
            pv = sc_ref[s]
            pv = jnp.where(tq >= BLOCK_SIZE * s + (BLOCK_SIZE + WINDOW - 1),
                           pv, neg_inf)
            v.append(pv)
            i.append(jnp.full((NC, 128), 1, jnp.int32) * s)
        for a, b in _PAIRS16:
            _ce(v, i, a, b)
        return v, i

    v0, i0 = sorted_group(0)

    def body(g, carry):
        cv, ci = list(carry[0]), list(carry[1])
        nv, ni = sorted_group(g)
        mv, mi = _merge_top16(cv, ci, nv, ni)
        return (tuple(mv), tuple(mi))

    fv, fi = jax.lax.fori_loop(1, l_idx + 1, body, (tuple(v0), tuple(i0)))
    fv, fi = list(fv), list(fi)
    for t in range(TOPK):
        vt = fv[t]
        it = jnp.where(vt == neg_inf, jnp.int32(-1), fi[t])
        ov_ref[0, 0, h * TOPK + t] = vt
        oi_ref[0, 0, h * TOPK + t] = it


@jax.jit
def kernel(q, k):
    k_scaled = k * jnp.float32(SM_SCALE)
    nl = L // LT
    grid = (B, nl, H)
    out_shape = (
        jax.ShapeDtypeStruct((B, nl, H * TOPK, NC, 128), jnp.float32),
        jax.ShapeDtypeStruct((B, nl, H * TOPK, NC, 128), jnp.int32),
    )
    q_spec = pl.BlockSpec((1, LT, H, D), lambda b, l, h: (b, l, 0, 0))
    k_spec = pl.BlockSpec((1, S, H, D), lambda b, l, h: (b, 0, 0, 0))
    o_spec = pl.BlockSpec((1, 1, H * TOPK, NC, 128), lambda b, l, h: (b, l, 0, 0, 0))
    ov, oi = pl.pallas_call(
        _topk_kernel,
        grid=grid,
        in_specs=[q_spec, k_spec],
        out_specs=(o_spec, o_spec),
        out_shape=out_shape,
        scratch_shapes=[pltpu.VMEM((S, NC, 128), jnp.float32)],
    )(q, k_scaled)
    ov = ov.transpose(0, 1, 3, 4, 2).reshape(B, L, H, TOPK)
    oi = oi.transpose(0, 1, 3, 4, 2).reshape(B, L, H, TOPK)
    return ov, oi
